# baseline (speedup 1.0000x reference)
import functools

import jax
import jax.numpy as jnp
from jax import lax
from jax.experimental import pallas as pl
from jax.experimental.pallas import tpu as pltpu
from jax.experimental.pallas import tpu_sc as plsc

DIM_Z = 64
N_INT = 65
WPR = 2
NC, NS = 1, 16
NW = NC * NS
LANES = 16


def _make_sc_lookup(batch: int):
    bpw = batch // NW
    wpw = bpw * WPR
    mesh = plsc.VectorSubcoreMesh(
        core_axis_name="c", subcore_axis_name="s", num_cores=1)

    @functools.partial(
        pl.kernel,
        mesh=mesh,
        out_type=jax.ShapeDtypeStruct((batch * WPR,), jnp.int32),
        scratch_types=[
            pltpu.VMEM((bpw,), jnp.int32),
            pltpu.VMEM((N_INT,), jnp.int32),
            pltpu.VMEM((N_INT * WPR,), jnp.int32),
            pltpu.VMEM((wpw,), jnp.int32),
            pltpu.SemaphoreType.DMA,
        ],
        compiler_params=pltpu.CompilerParams(
            needs_layout_passes=False, use_tc_tiling_on_sc=False,
            skip_device_barrier=True),
    )
    def sc_lookup(labels_hbm, perm_hbm, masks_hbm, out_hbm,
                  labels_v, perm_v, table_v, rows_v, sem):
        wid = lax.axis_index("s") * NC + lax.axis_index("c")
        base = wid * bpw
        ins = [pltpu.async_copy(labels_hbm.at[pl.ds(base, bpw)], labels_v, sem),
               pltpu.async_copy(perm_hbm, perm_v, sem),
               pltpu.async_copy(masks_hbm, table_v, sem)]
        for c in ins:
            c.wait()
        lanes = lax.iota(jnp.int32, LANES)

        @plsc.parallel_loop(0, bpw // LANES, 1, unroll=4)
        def _groups(i):
            lbl = labels_v[pl.ds(i * LANES, LANES)]
            idx = plsc.load_gather(perm_v, [lbl])
            idx2 = idx * WPR
            pos = lanes + i * LANES
            for w in range(WPR):
                vals = plsc.load_gather(table_v, [idx2 + w])
                plsc.store_scatter(rows_v, [pos + w * bpw], vals)

        outs = [pltpu.async_copy(rows_v.at[pl.ds(0, bpw)],
                                 out_hbm.at[pl.ds(base, bpw)], sem),
                pltpu.async_copy(rows_v.at[pl.ds(bpw, bpw)],
                                 out_hbm.at[pl.ds(batch + base, bpw)], sem)]
        for c in outs:
            c.wait()

    return sc_lookup


def kernel(intervention_label, permutation, masks):
    batch = intervention_label.shape[0]
    bits = masks.reshape(N_INT * WPR, 32).astype(jnp.uint32)
    table = (bits << jnp.arange(32, dtype=jnp.uint32)).sum(
        axis=1, dtype=jnp.uint32).view(jnp.int32)
    words = _make_sc_lookup(batch)(intervention_label, permutation, table)
    lo, hi = words[:batch, None], words[batch:, None]
    cols = jnp.arange(DIM_Z, dtype=jnp.int32)[None, :]
    sel = jnp.where(cols < 32, lo, hi)
    return ((sel >> (cols & 31)) & 1) != 0
